# Initial kernel scaffold; baseline (speedup 1.0000x reference)
import jax
import jax.numpy as jnp
from jax.experimental import pallas as pl


def kernel(x, W1, b1, W2, b2, Wr, A, B, scale):
    raise NotImplementedError("write your pallas kernel here")


# speedup vs baseline: 2.7169x; 2.7169x over previous
import functools

import jax
import jax.numpy as jnp
from jax.experimental import pallas as pl

T = 4096
D_MODEL = 1024
D_FF = 4096
E = 8
TOPK = 2
R = 64

TILE_T = 256


def _moe_kernel(x_ref, w1_ref, b1_ref, w2_ref, b2_ref, wr_ref, a_ref, b_lora_ref,
                scale_ref, out_ref):
    x = x_ref[...]
    h = jax.nn.gelu(
        jnp.dot(x, w1_ref[...], preferred_element_type=jnp.float32) + b1_ref[...]
    )
    base = jnp.dot(h, w2_ref[...], preferred_element_type=jnp.float32) + b2_ref[...]

    logits = jnp.dot(x, wr_ref[...], preferred_element_type=jnp.float32)
    probs = jax.nn.softmax(logits, axis=-1)
    ids = jax.lax.broadcasted_iota(jnp.int32, probs.shape, 1)
    v1 = jnp.max(probs, axis=-1, keepdims=True)
    i1 = jnp.min(jnp.where(probs == v1, ids, E), axis=-1, keepdims=True)
    m1 = ids == i1
    probs2 = jnp.where(m1, -1.0, probs)
    v2 = jnp.max(probs2, axis=-1, keepdims=True)
    i2 = jnp.min(jnp.where(probs2 == v2, ids, E), axis=-1, keepdims=True)
    m2 = ids == i2
    denom = v1 + v2 + 1e-9
    w = (jnp.where(m1, v1, 0.0) + jnp.where(m2, v2, 0.0)) / denom
    wscale = w * scale_ref[...]

    low = jnp.dot(h, a_ref[...], preferred_element_type=jnp.float32)
    col_e = jax.lax.broadcasted_iota(jnp.int32, low.shape, 1) // R
    gm = jnp.zeros_like(low)
    for e in range(E):
        gm = jnp.where(col_e == e, wscale[:, e:e + 1], gm)
    moe = jnp.dot(low * gm, b_lora_ref[...], preferred_element_type=jnp.float32)

    out_ref[...] = base + moe


@jax.jit
def kernel(x, W1, b1, W2, b2, Wr, A, B, scale):
    a2d = jnp.transpose(A, (1, 0, 2)).reshape(D_FF, E * R)
    b2d = B.reshape(E * R, D_MODEL)
    grid = (T // TILE_T,)
    full = lambda i: (0, 0)
    out = pl.pallas_call(
        _moe_kernel,
        grid=grid,
        in_specs=[
            pl.BlockSpec((TILE_T, D_MODEL), lambda i: (i, 0)),
            pl.BlockSpec((D_MODEL, D_FF), full),
            pl.BlockSpec((1, D_FF), full),
            pl.BlockSpec((D_FF, D_MODEL), full),
            pl.BlockSpec((1, D_MODEL), full),
            pl.BlockSpec((D_MODEL, E), full),
            pl.BlockSpec((D_FF, E * R), full),
            pl.BlockSpec((E * R, D_MODEL), full),
            pl.BlockSpec((1, E), full),
        ],
        out_specs=pl.BlockSpec((TILE_T, D_MODEL), lambda i: (i, 0)),
        out_shape=jax.ShapeDtypeStruct((T, D_MODEL), jnp.float32),
    )(x, W1, b1.reshape(1, D_FF), W2, b2.reshape(1, D_MODEL), Wr,
      a2d, b2d, scale.reshape(1, E))
    return out
